# baseline (speedup 1.0000x reference)
import functools

import jax
import jax.numpy as jnp
from jax import lax
from jax.experimental import pallas as pl
from jax.experimental.pallas import tpu as pltpu
from jax.experimental.pallas import tpu_sc as plsc

NC = 2
NS = 16
NW = NC * NS


def _h_matrix(z):
    n = z.shape[0]
    bi = 80
    assert n % bi == 0

    def body(zi_ref, zj_ref, out_ref, nj_ref):
        zj = zj_ref[...]

        @pl.when(pl.program_id(0) == 0)
        def _():
            zz = zj * zj
            ones = jnp.ones((1, zj.shape[1]), jnp.float32)
            nj_ref[...] = lax.dot_general(ones, zz, (((1,), (1,)), ((), ())),
                                          preferred_element_type=jnp.float32)

        a = zi_ref[...].astype(jnp.bfloat16)
        b = zj.astype(jnp.bfloat16)
        g = lax.dot_general(a, b, (((1,), (1,)), ((), ())),
                            preferred_element_type=jnp.float32)
        out_ref[...] = nj_ref[...] - 2.0 * g

    return pl.pallas_call(
        body,
        grid=(n // bi,),
        in_specs=[
            pl.BlockSpec((bi, z.shape[1]), lambda i: (i, 0)),
            pl.BlockSpec((n, z.shape[1]), lambda i: (0, 0)),
        ],
        out_specs=pl.BlockSpec((bi, n), lambda i: (i, 0)),
        out_shape=jax.ShapeDtypeStruct((n, n), jnp.float32),
        scratch_shapes=[pltpu.VMEM((1, n), jnp.float32)],
    )(z, z)


def _make_scatter(nn, ewp, cl):
    ncha = ewp // cl
    ngrp = cl // 128
    mesh = plsc.VectorSubcoreMesh(core_axis_name="c", subcore_axis_name="s",
                                  num_cores=NC, num_subcores=NS)

    @functools.partial(
        pl.kernel, mesh=mesh, out_type=(),
        scratch_types=[
            pltpu.VMEM((2, 2, cl), jnp.int32),
            pltpu.VMEM((2, ngrp, 128), jnp.int32),
            pltpu.VMEM((128,), jnp.int32),
            pltpu.SemaphoreType.DMA, pltpu.SemaphoreType.DMA,
            pltpu.SemaphoreType.DMA, pltpu.SemaphoreType.DMA,
        ],
    )
    def scatter_kernel(map_hbm, ls_hbm, lsv, qsv, onesv,
                       sem_l0, sem_l1, sem_s0, sem_s1):
        sem_l = (sem_l0, sem_l1)
        sem_s = (sem_s0, sem_s1)
        wid = lax.axis_index("s") * NC + lax.axis_index("c")
        wbase = wid * ewp
        off = jnp.where(wid < NW // 2, 0, nn * nn).astype(jnp.int32)

        for ss in range(8):
            onesv[pl.ds(ss * 16, 16)] = jnp.full((16,), 1, jnp.int32)

        def fire_lin(m, slot):
            pltpu.async_copy(ls_hbm.at[:, pl.ds(wbase + m * cl, cl)],
                             lsv.at[slot], sem_l[slot])

        def wait_old(slot):
            for g in range(ngrp):
                pltpu.make_async_copy(
                    onesv, map_hbm.at[qsv.at[slot, g]], sem_s[slot]).wait()

        def proc(p, slot):
            pltpu.make_async_copy(ls_hbm.at[:, pl.ds(0, cl)], lsv.at[slot],
                                  sem_l[slot]).wait()
            pl.when(p >= 2)(lambda: wait_old(slot))
            for g in range(ngrp):
                for ss in range(8):
                    sl = pl.ds(g * 128 + ss * 16, 16)
                    i16 = lsv[slot, 0, sl]
                    j16 = lsv[slot, 1, sl]
                    qsv[slot, g, pl.ds(ss * 16, 16)] = i16 * nn + j16 + off
            for g in range(ngrp):
                pltpu.async_copy(onesv, map_hbm.at[qsv.at[slot, g]],
                                 sem_s[slot])

        t_a = ncha + 1
        t_a += t_a % 2

        @pl.loop(0, t_a, step=2)
        def _(b2):
            for b in range(2):
                m = b2 + b
                pl.when(m < ncha)(lambda m=m, s=b: fire_lin(m, s))
                pl.when((m >= 1) & (m <= ncha))(
                    lambda m=m, s=(b + 1) % 2: proc(m - 1, s))

        for slot in range(2):
            wait_old(slot)

    return scatter_kernel


def _make_compute(nn, ewp, e_total):
    c = 128
    nch = ewp // c
    nb = 6
    t_total = nch + nb
    t_total += (-t_total) % nb
    mesh = plsc.VectorSubcoreMesh(core_axis_name="c", subcore_axis_name="s",
                                  num_cores=NC, num_subcores=NS)

    @functools.partial(
        pl.kernel, mesh=mesh,
        out_type=jax.ShapeDtypeStruct((NW, 16), jnp.float32),
        scratch_types=[
            pltpu.VMEM((nb, 6, c), jnp.int32),
            pltpu.VMEM((nb, 4, c), jnp.int32),
            pltpu.VMEM((nb, c), jnp.int32),
            pltpu.VMEM((nb, 4, c), jnp.int32),
            pltpu.VMEM((nb, c), jnp.float32),
            pltpu.VMEM((nb, c), jnp.int32),
            pltpu.VMEM((nb, c), jnp.float32),
            pltpu.VMEM((16,), jnp.float32),
        ] + [pltpu.SemaphoreType.DMA] * (3 * nb),
    )
    def compute_kernel(map_hbm, h_hbm, lz_hbm, out_hbm,
                       lv, qv, qjv, mv, gjv, qkv, gkv, accv, *sems):
        sem_l = sems[0:nb]
        sem_m = sems[nb:2 * nb]
        sem_g = sems[2 * nb:3 * nb]
        wid = lax.axis_index("s") * NC + lax.axis_index("c")
        wbase = wid * ewp
        off = jnp.where(wid < NW // 2, 0, nn * nn).astype(jnp.int32)
        sgn = jnp.where(wid < NW // 2, 1.0, -1.0).astype(jnp.float32)

        accv[...] = jnp.zeros((16,), jnp.float32)

        def stage_a(m, slot):
            pltpu.async_copy(lz_hbm.at[:, pl.ds(wbase + m * c, c)],
                             lv.at[slot], sem_l[slot])

        def stage_b(m, slot):
            pltpu.make_async_copy(lz_hbm.at[:, pl.ds(0, c)], lv.at[slot],
                                  sem_l[slot]).wait()
            for ss in range(8):
                sl = pl.ds(ss * 16, 16)
                i16 = lv[slot, 0, sl]
                inn = i16 * nn
                qjv[slot, sl] = inn + lv[slot, 1, sl]
                for t in range(4):
                    qv[slot, t, sl] = inn + lv[slot, 2 + t, sl] + off
            for t in range(4):
                pltpu.async_copy(map_hbm.at[qv.at[slot, t]], mv.at[slot, t],
                                 sem_m[slot])
            pltpu.async_copy(h_hbm.at[qjv.at[slot]], gjv.at[slot],
                             sem_m[slot])

        def stage_c(m, slot):
            for t in range(4):
                pltpu.make_async_copy(map_hbm.at[qv.at[slot, t]],
                                      mv.at[slot, t], sem_m[slot]).wait()
            pltpu.make_async_copy(h_hbm.at[qjv.at[slot]], gjv.at[slot],
                                  sem_m[slot]).wait()
            for ss in range(8):
                sl = pl.ds(ss * 16, 16)
                i16 = lv[slot, 0, sl]
                k16 = lv[slot, 2, sl]
                m16 = mv[slot, 0, sl]
                for t in range(1, 4):
                    hit = m16 != 0
                    k16 = jnp.where(hit, lv[slot, 2 + t, sl], k16)
                    m16 = jnp.where(hit, mv[slot, t, sl], m16)
                qkv[slot, sl] = i16 * nn + k16
            pltpu.async_copy(h_hbm.at[qkv.at[slot]], gkv.at[slot],
                             sem_g[slot])

        def stage_d(m, slot):
            pltpu.make_async_copy(h_hbm.at[qkv.at[slot]], gkv.at[slot],
                                  sem_g[slot]).wait()
            for ss in range(8):
                sl = pl.ds(ss * 16, 16)
                s16 = sgn * (gjv[slot, sl] - gkv[slot, sl])
                accv[...] = accv[...] + jnp.maximum(s16, 0.0)

        @pl.loop(0, t_total, step=nb)
        def _(base_it):
            for b in range(nb):
                m = base_it + b
                pl.when(m < nch)(lambda m=m, s=b: stage_a(m, s))
                pl.when((m >= 2) & (m < nch + 2))(
                    lambda m=m, s=(b - 2) % nb: stage_b(m - 2, s))
                pl.when((m >= 4) & (m < nch + 4))(
                    lambda m=m, s=(b - 4) % nb: stage_c(m - 4, s))
                pl.when((m >= 6) & (m < nch + 6))(
                    lambda m=m, s=b % nb: stage_d(m - 6, s))

        pltpu.sync_copy(accv, out_hbm.at[wid])

    return compute_kernel


def kernel(z, pos_edges, neg_edges):
    nn = z.shape[0]
    e = pos_edges.shape[1]
    ew = e // (NW // 2)
    ewp = ew + (-ew) % 1280

    key = jax.random.key(42)

    def draws(loss_id):
        k = jax.random.fold_in(key, loss_id)
        return [jax.random.randint(jax.random.fold_in(k, t), (e,), 0, nn,
                                   dtype=jnp.int32) for t in range(4)]

    kc_p = draws(1)
    kc_n = draws(2)

    def pad_blocks(x, fill):
        buf = jnp.full((NW // 2, ewp), fill, jnp.int32)
        return buf.at[:, :ew].set(x.reshape(NW // 2, ew)).reshape(-1)

    ip, jp = pos_edges[0].astype(jnp.int32), pos_edges[1].astype(jnp.int32)
    im, jm = neg_edges[0].astype(jnp.int32), neg_edges[1].astype(jnp.int32)

    lz = jnp.stack(
        [jnp.concatenate([pad_blocks(ip, 0), pad_blocks(im, 0)])]
        + [jnp.concatenate([pad_blocks(jp, 0), pad_blocks(jm, 0)])]
        + [jnp.concatenate([pad_blocks(kc_p[t], 0), pad_blocks(kc_n[t], 0)])
           for t in range(4)])

    i_s = jnp.concatenate([pad_blocks(ip, 2 * nn), pad_blocks(im, nn)])
    j_s = jnp.concatenate([pad_blocks(jp, 0), pad_blocks(jm, 0)])
    ls = jnp.stack([i_s, j_s])

    h = _h_matrix(z).reshape(-1)

    map_ref = jax.new_ref(jnp.zeros(2 * nn * nn + 8, jnp.int32))
    _make_scatter(nn, ewp, 1280)(map_ref, ls)
    parts = _make_compute(nn, ewp, 2 * e)(map_ref, h, lz)

    return jnp.sum(parts) / jnp.float32(e)


# speedup vs baseline: 1.0185x; 1.0185x over previous
import functools

import jax
import jax.numpy as jnp
from jax import lax
from jax.experimental import pallas as pl
from jax.experimental.pallas import tpu as pltpu
from jax.experimental.pallas import tpu_sc as plsc

NC = 2
NS = 16
NW = NC * NS


def _h_matrix(z):
    n = z.shape[0]
    bi = 80
    assert n % bi == 0

    def body(zi_ref, zj_ref, out_ref, nj_ref):
        zj = zj_ref[...]

        @pl.when(pl.program_id(0) == 0)
        def _():
            zz = zj * zj
            ones = jnp.ones((1, zj.shape[1]), jnp.float32)
            nj_ref[...] = lax.dot_general(ones, zz, (((1,), (1,)), ((), ())),
                                          preferred_element_type=jnp.float32)

        a = zi_ref[...].astype(jnp.bfloat16)
        b = zj.astype(jnp.bfloat16)
        g = lax.dot_general(a, b, (((1,), (1,)), ((), ())),
                            preferred_element_type=jnp.float32)
        out_ref[...] = nj_ref[...] - 2.0 * g

    return pl.pallas_call(
        body,
        grid=(n // bi,),
        in_specs=[
            pl.BlockSpec((bi, z.shape[1]), lambda i: (i, 0)),
            pl.BlockSpec((n, z.shape[1]), lambda i: (0, 0)),
        ],
        out_specs=pl.BlockSpec((bi, n), lambda i: (i, 0)),
        out_shape=jax.ShapeDtypeStruct((n, n), jnp.float32),
        scratch_shapes=[pltpu.VMEM((1, n), jnp.float32)],
    )(z, z)


def _make_scatter(nn, ewp, cl):
    ncha = ewp // cl
    ngrp = cl // 128
    mesh = plsc.VectorSubcoreMesh(core_axis_name="c", subcore_axis_name="s",
                                  num_cores=NC, num_subcores=NS)

    @functools.partial(
        pl.kernel, mesh=mesh, out_type=(),
        scratch_types=[
            pltpu.VMEM((2, 2, cl), jnp.int32),
            pltpu.VMEM((2, ngrp, 128), jnp.int32),
            pltpu.VMEM((128,), jnp.int32),
            pltpu.SemaphoreType.DMA, pltpu.SemaphoreType.DMA,
            pltpu.SemaphoreType.DMA, pltpu.SemaphoreType.DMA,
        ],
    )
    def scatter_kernel(map_hbm, ls_hbm, lsv, qsv, onesv,
                       sem_l0, sem_l1, sem_s0, sem_s1):
        sem_l = (sem_l0, sem_l1)
        sem_s = (sem_s0, sem_s1)
        wid = lax.axis_index("s") * NC + lax.axis_index("c")
        wbase = wid * ewp
        off = jnp.where(wid < NW // 2, 0, nn * nn).astype(jnp.int32)

        for ss in range(8):
            onesv[pl.ds(ss * 16, 16)] = jnp.full((16,), 1, jnp.int32)

        def fire_lin(m, slot):
            pltpu.async_copy(ls_hbm.at[:, pl.ds(wbase + m * cl, cl)],
                             lsv.at[slot], sem_l[slot])

        def wait_old(slot):
            for g in range(ngrp):
                pltpu.make_async_copy(
                    onesv, map_hbm.at[qsv.at[slot, g]], sem_s[slot]).wait()

        def proc(p, slot):
            pltpu.make_async_copy(ls_hbm.at[:, pl.ds(0, cl)], lsv.at[slot],
                                  sem_l[slot]).wait()
            pl.when(p >= 2)(lambda: wait_old(slot))
            for g in range(ngrp):
                for ss in range(8):
                    sl = pl.ds(g * 128 + ss * 16, 16)
                    i16 = lsv[slot, 0, sl]
                    j16 = lsv[slot, 1, sl]
                    qsv[slot, g, pl.ds(ss * 16, 16)] = i16 * nn + j16 + off
            for g in range(ngrp):
                pltpu.async_copy(onesv, map_hbm.at[qsv.at[slot, g]],
                                 sem_s[slot])

        t_a = ncha + 1
        t_a += t_a % 2

        @pl.loop(0, t_a, step=2)
        def _(b2):
            for b in range(2):
                m = b2 + b
                pl.when(m < ncha)(lambda m=m, s=b: fire_lin(m, s))
                pl.when((m >= 1) & (m <= ncha))(
                    lambda m=m, s=(b + 1) % 2: proc(m - 1, s))

        for slot in range(2):
            wait_old(slot)

    return scatter_kernel


def _make_compute(nn, ewp, e_total):
    c = 128
    nch = ewp // c
    nb = 6
    t_total = nch + nb
    t_total += (-t_total) % nb
    mesh = plsc.VectorSubcoreMesh(core_axis_name="c", subcore_axis_name="s",
                                  num_cores=NC, num_subcores=NS)

    @functools.partial(
        pl.kernel, mesh=mesh,
        out_type=jax.ShapeDtypeStruct((NW, 16), jnp.float32),
        scratch_types=[
            pltpu.VMEM((nb, 6, c), jnp.int32),
            pltpu.VMEM((nb, 3, c), jnp.int32),
            pltpu.VMEM((nb, c), jnp.int32),
            pltpu.VMEM((nb, 3, c), jnp.int32),
            pltpu.VMEM((nb, c), jnp.float32),
            pltpu.VMEM((nb, c), jnp.int32),
            pltpu.VMEM((nb, c), jnp.float32),
            pltpu.VMEM((16,), jnp.float32),
        ] + [pltpu.SemaphoreType.DMA] * (3 * nb),
    )
    def compute_kernel(map_hbm, h_hbm, lz_hbm, out_hbm,
                       lv, qv, qjv, mv, gjv, qkv, gkv, accv, *sems):
        sem_l = sems[0:nb]
        sem_m = sems[nb:2 * nb]
        sem_g = sems[2 * nb:3 * nb]
        wid = lax.axis_index("s") * NC + lax.axis_index("c")
        wbase = wid * ewp
        off = jnp.where(wid < NW // 2, 0, nn * nn).astype(jnp.int32)
        sgn = jnp.where(wid < NW // 2, 1.0, -1.0).astype(jnp.float32)

        accv[...] = jnp.zeros((16,), jnp.float32)

        def any_hit(slot):
            h16 = mv[slot, 0, pl.ds(0, 16)]
            for ss in range(1, c // 16):
                h16 = h16 | mv[slot, 0, pl.ds(ss * 16, 16)]
            return jnp.max(h16) != 0

        def stage_a(m, slot):
            pltpu.async_copy(lz_hbm.at[:, pl.ds(wbase + m * c, c)],
                             lv.at[slot], sem_l[slot])

        def stage_b(m, slot):
            pltpu.make_async_copy(lz_hbm.at[:, pl.ds(0, c)], lv.at[slot],
                                  sem_l[slot]).wait()
            for ss in range(c // 16):
                sl = pl.ds(ss * 16, 16)
                i16 = lv[slot, 0, sl]
                inn = i16 * nn
                qjv[slot, sl] = inn + lv[slot, 1, sl]
                qkv[slot, sl] = inn + lv[slot, 2, sl]
                for t in range(3):
                    qv[slot, t, sl] = inn + lv[slot, 2 + t, sl] + off
            pltpu.async_copy(map_hbm.at[qv.at[slot, 0]], mv.at[slot, 0],
                             sem_m[slot])
            pltpu.async_copy(h_hbm.at[qjv.at[slot]], gjv.at[slot],
                             sem_m[slot])

        def stage_c(m, slot):
            pltpu.make_async_copy(map_hbm.at[qv.at[slot, 0]], mv.at[slot, 0],
                                  sem_m[slot]).wait()
            pltpu.make_async_copy(h_hbm.at[qjv.at[slot]], gjv.at[slot],
                                  sem_m[slot]).wait()
            for t in (1, 2):
                pltpu.async_copy(map_hbm.at[qv.at[slot, t]],
                                 mv.at[slot, t], sem_m[slot])

        def stage_d(m, slot):
            if True:
                for t in (1, 2):
                    pltpu.make_async_copy(map_hbm.at[qv.at[slot, t]],
                                          mv.at[slot, t], sem_m[slot]).wait()
                for ss in range(c // 16):
                    sl = pl.ds(ss * 16, 16)
                    hit0 = mv[slot, 0, sl] != 0
                    k16 = jnp.where(hit0, lv[slot, 3, sl], lv[slot, 2, sl])
                    m16 = jnp.where(hit0, mv[slot, 1, sl], 0)
                    hit1 = m16 != 0
                    k16 = jnp.where(hit1, lv[slot, 4, sl], k16)
                    m16 = jnp.where(hit1, mv[slot, 2, sl], 0)
                    k16 = jnp.where(m16 != 0, lv[slot, 5, sl], k16)
                    qkv[slot, sl] = lv[slot, 0, sl] * nn + k16
            pltpu.async_copy(h_hbm.at[qkv.at[slot]], gkv.at[slot],
                             sem_g[slot])

        def stage_e(m, slot):
            pltpu.make_async_copy(h_hbm.at[qkv.at[slot]], gkv.at[slot],
                                  sem_g[slot]).wait()
            for ss in range(c // 16):
                sl = pl.ds(ss * 16, 16)
                s16 = sgn * (gjv[slot, sl] - gkv[slot, sl])
                accv[...] = accv[...] + jnp.maximum(s16, 0.0)

        @pl.loop(0, t_total, step=nb)
        def _(base_it):
            for b in range(nb):
                m = base_it + b
                pl.when(m < nch)(lambda m=m, s=b: stage_a(m, s))
                pl.when((m >= 2) & (m < nch + 2))(
                    lambda m=m, s=(b - 2) % nb: stage_b(m - 2, s))
                pl.when((m >= 4) & (m < nch + 4))(
                    lambda m=m, s=(b - 4) % nb: stage_c(m - 4, s))
                pl.when((m >= 5) & (m < nch + 5))(
                    lambda m=m, s=(b - 5) % nb: stage_d(m - 5, s))
                pl.when((m >= 6) & (m < nch + 6))(
                    lambda m=m, s=b % nb: stage_e(m - 6, s))

        pltpu.sync_copy(accv, out_hbm.at[wid])

    return compute_kernel


def kernel(z, pos_edges, neg_edges):
    nn = z.shape[0]
    e = pos_edges.shape[1]
    ew = e // (NW // 2)
    ewp = ew + (-ew) % 1280

    key = jax.random.key(42)

    def draws(loss_id):
        k = jax.random.fold_in(key, loss_id)
        return [jax.random.randint(jax.random.fold_in(k, t), (e,), 0, nn,
                                   dtype=jnp.int32) for t in range(4)]

    kc_p = draws(1)
    kc_n = draws(2)

    def pad_blocks(x, fill):
        buf = jnp.full((NW // 2, ewp), fill, jnp.int32)
        return buf.at[:, :ew].set(x.reshape(NW // 2, ew)).reshape(-1)

    ip, jp = pos_edges[0].astype(jnp.int32), pos_edges[1].astype(jnp.int32)
    im, jm = neg_edges[0].astype(jnp.int32), neg_edges[1].astype(jnp.int32)

    lz = jnp.stack(
        [jnp.concatenate([pad_blocks(ip, 0), pad_blocks(im, 0)])]
        + [jnp.concatenate([pad_blocks(jp, 0), pad_blocks(jm, 0)])]
        + [jnp.concatenate([pad_blocks(kc_p[t], 0), pad_blocks(kc_n[t], 0)])
           for t in range(4)])

    i_s = jnp.concatenate([pad_blocks(ip, 2 * nn), pad_blocks(im, nn)])
    j_s = jnp.concatenate([pad_blocks(jp, 0), pad_blocks(jm, 0)])
    ls = jnp.stack([i_s, j_s])

    h = _h_matrix(z).reshape(-1)

    map_ref = jax.new_ref(jnp.zeros(2 * nn * nn + 8, jnp.int32))
    _make_scatter(nn, ewp, 1280)(map_ref, ls)
    parts = _make_compute(nn, ewp, 2 * e)(map_ref, h, lz)

    return jnp.sum(parts) / jnp.float32(e)
